# baseline (speedup 1.0000x reference)
import functools

import jax
import jax.numpy as jnp
from jax import lax
from jax.experimental import pallas as pl
from jax.experimental.pallas import tpu as pltpu

LEVELS = (8, 6, 5)


def _shift3(a, zcol):
    am = jnp.concatenate([zcol, a[:, :-1]], axis=1)
    ap = jnp.concatenate([a[:, 1:], zcol], axis=1)
    return jnp.concatenate([am, a, ap], axis=0)


def _fsq_body(x_ref, w1_ref, b1_ref, w2_ref, b2_ref, w3_ref, b3_ref,
              wp_ref, bp_ref, half_ref, inv_ref, codes_ref, out_ref,
              probs_ref, perp_ref, acc_ref, *, nsteps, total):
    step = pl.program_id(0)

    @pl.when(step == 0)
    def _init():
        acc_ref[:, :] = jnp.zeros((32, 8), jnp.float32)

    T = x_ref.shape[-1]
    ci = x_ref.shape[1]
    half = half_ref[:, :]
    inv = inv_ref[:, :]

    xb = x_ref[0]
    x3 = _shift3(xb, jnp.zeros((ci, 1), jnp.float32))
    h1 = jnp.maximum(
        jnp.dot(w1_ref[:, :], x3, preferred_element_type=jnp.float32)
        + b1_ref[:, :], 0.0)
    h3 = _shift3(h1, jnp.zeros((h1.shape[0], 1), jnp.float32))
    h2 = jnp.maximum(
        jnp.dot(w2_ref[:, :], h3, preferred_element_type=jnp.float32)
        + b2_ref[:, :], 0.0)
    f = jnp.dot(w3_ref[:, :], h2,
                preferred_element_type=jnp.float32) + b3_ref[:, :]
    tb = jnp.tanh(f)
    idxf = jnp.round((tb + 1.0) * half)
    q = idxf * inv - 1.0
    wpf = wp_ref[:, :]
    ob = bp_ref[:, :]
    for d in range(3):
        ob = ob + wpf[:, d:d + 1] * q[d:d + 1, :]
    out_ref[0] = ob
    idxi = idxf.astype(jnp.int32)
    i0 = idxi[0:1]
    codes_ref[0] = i0 + idxi[1:2] * 8 + idxi[2:3] * 48
    j12 = idxi[1:2] + idxi[2:3] * 6
    oh8 = (lax.broadcasted_iota(jnp.int32, (8, T), 0) == i0
           ).astype(jnp.bfloat16)
    oh30 = (lax.broadcasted_iota(jnp.int32, (32, T), 0) == j12
            ).astype(jnp.bfloat16)
    acc_ref[:, :] += lax.dot_general(
        oh30, oh8, (((1,), (1,)), ((), ())),
        preferred_element_type=jnp.float32)

    @pl.when(step == nsteps - 1)
    def _fin():
        counts = acc_ref[:, :]
        probs = counts * (1.0 / total)
        pos = probs > 0
        ent = -jnp.sum(jnp.where(pos, probs * jnp.log(
            jnp.where(pos, probs, 1.0)), 0.0))
        probs_ref[:, :] = probs
        perp_ref[:, :] = jnp.maximum(jnp.exp(ent), 1.0)[None, None]


def kernel(x, w1, b1, w2, b2, w3, b3, wp, bp):
    B, C, S, T = x.shape
    ci = C * S
    xr = x.reshape(B, ci, T)
    nsteps = B
    total = B * T

    w1f = w1.transpose(0, 2, 1).reshape(w1.shape[0], -1)
    w2f = w2.transpose(0, 2, 1).reshape(w2.shape[0], -1)
    w3f = jnp.pad(w3[:, :, 0], ((0, 5), (0, 0)))
    wpf = jnp.pad(wp[:, :, 0], ((0, 0), (0, 5)))
    b1c = b1[:, None]
    b2c = b2[:, None]
    b3c = jnp.pad(b3[:, None], ((0, 5), (0, 0)))
    bpc = bp[:, None]
    half = jnp.array([[3.5], [2.5], [2.0], [0.], [0.], [0.], [0.], [0.]],
                     jnp.float32)
    inv = jnp.array([[2.0 / 7.0], [2.0 / 5.0], [0.5], [0.], [0.], [0.],
                     [0.], [0.]], jnp.float32)

    body = functools.partial(_fsq_body, nsteps=nsteps, total=float(total))
    codes3, out, probs32x8, perp11 = pl.pallas_call(
        body,
        grid=(nsteps,),
        in_specs=[
            pl.BlockSpec((1, ci, T), lambda i: (i, 0, 0)),
            pl.BlockSpec(w1f.shape, lambda i: (0, 0)),
            pl.BlockSpec(b1c.shape, lambda i: (0, 0)),
            pl.BlockSpec(w2f.shape, lambda i: (0, 0)),
            pl.BlockSpec(b2c.shape, lambda i: (0, 0)),
            pl.BlockSpec(w3f.shape, lambda i: (0, 0)),
            pl.BlockSpec(b3c.shape, lambda i: (0, 0)),
            pl.BlockSpec(wpf.shape, lambda i: (0, 0)),
            pl.BlockSpec(bpc.shape, lambda i: (0, 0)),
            pl.BlockSpec((8, 1), lambda i: (0, 0)),
            pl.BlockSpec((8, 1), lambda i: (0, 0)),
        ],
        out_specs=[
            pl.BlockSpec((1, 1, T), lambda i: (i, 0, 0)),
            pl.BlockSpec((1, 64, T), lambda i: (i, 0, 0)),
            pl.BlockSpec((32, 8), lambda i: (0, 0)),
            pl.BlockSpec((1, 1), lambda i: (0, 0)),
        ],
        out_shape=[
            jax.ShapeDtypeStruct((B, 1, T), jnp.int32),
            jax.ShapeDtypeStruct((B, 64, T), jnp.float32),
            jax.ShapeDtypeStruct((32, 8), jnp.float32),
            jax.ShapeDtypeStruct((1, 1), jnp.float32),
        ],
        scratch_shapes=[pltpu.VMEM((32, 8), jnp.float32)],
        compiler_params=pltpu.CompilerParams(
            dimension_semantics=("arbitrary",)),
    )(xr, w1f, b1c, w2f, b2c, w3f, b3c, wpf, bpc, half, inv)

    codes = codes3.reshape(B, T)
    probs = probs32x8.reshape(-1)[:240]
    perplexity = perp11.reshape(())
    return codes, out, perplexity, probs


# speedup vs baseline: 1.2375x; 1.2375x over previous
import functools

import jax
import jax.numpy as jnp
from jax import lax
from jax.experimental import pallas as pl
from jax.experimental.pallas import tpu as pltpu

LEVELS = (8, 6, 5)
NB = 8


def _shift3(a, zcol):
    am = jnp.concatenate([zcol, a[:, :-1]], axis=1)
    ap = jnp.concatenate([a[:, 1:], zcol], axis=1)
    return jnp.concatenate([am, a, ap], axis=0)


def _fsq_body(x_ref, w1_ref, b1_ref, w2_ref, b2_ref, w3_ref, b3_ref,
              wp_ref, bp_ref, half_ref, inv_ref, codes_ref, out_ref,
              probs_ref, perp_ref, acc_ref, *, nsteps, total):
    step = pl.program_id(0)

    @pl.when(step == 0)
    def _init():
        acc_ref[:, :] = jnp.zeros((32, 8), jnp.float32)

    T = x_ref.shape[-1]
    ci = x_ref.shape[1]
    half = half_ref[:, :]
    inv = inv_ref[:, :]

    wpf = wp_ref[:, :]
    acc = jnp.zeros((32, 8), jnp.float32)
    code_rows = []
    for b in range(NB):
        xb = x_ref[b]
        x3 = _shift3(xb, jnp.zeros((ci, 1), jnp.float32))
        h1 = jnp.maximum(
            jnp.dot(w1_ref[:, :], x3, preferred_element_type=jnp.float32)
            + b1_ref[:, :], 0.0)
        h3 = _shift3(h1, jnp.zeros((h1.shape[0], 1), jnp.float32))
        h2 = jnp.maximum(
            jnp.dot(w2_ref[:, :], h3, preferred_element_type=jnp.float32)
            + b2_ref[:, :], 0.0)
        f = jnp.dot(w3_ref[:, :], h2,
                    preferred_element_type=jnp.float32) + b3_ref[:, :]
        tb = jnp.tanh(f)
        idxf = jnp.round((tb + 1.0) * half)
        q = idxf * inv - 1.0
        ob = bp_ref[:, :]
        for d in range(3):
            ob = ob + wpf[:, d:d + 1] * q[d:d + 1, :]
        out_ref[b] = ob
        idxi = idxf.astype(jnp.int32)
        i0 = idxi[0:1]
        code_rows.append(i0 + idxi[1:2] * 8 + idxi[2:3] * 48)
        j12 = idxi[1:2] + idxi[2:3] * 6
        oh8 = (lax.broadcasted_iota(jnp.int32, (8, T), 0) == i0
               ).astype(jnp.bfloat16)
        oh30 = (lax.broadcasted_iota(jnp.int32, (32, T), 0) == j12
                ).astype(jnp.bfloat16)
        acc = acc + lax.dot_general(
            oh30, oh8, (((1,), (1,)), ((), ())),
            preferred_element_type=jnp.float32)
    codes_ref[:, :] = jnp.concatenate(code_rows, axis=0)
    acc_ref[:, :] += acc

    @pl.when(step == nsteps - 1)
    def _fin():
        counts = acc_ref[:, :]
        probs = counts * (1.0 / total)
        pos = probs > 0
        ent = -jnp.sum(jnp.where(pos, probs * jnp.log(
            jnp.where(pos, probs, 1.0)), 0.0))
        probs_ref[:, :] = probs
        perp_ref[:, :] = jnp.maximum(jnp.exp(ent), 1.0)[None, None]


def kernel(x, w1, b1, w2, b2, w3, b3, wp, bp):
    B, C, S, T = x.shape
    ci = C * S
    xr = x.reshape(B, ci, T)
    nsteps = B // NB
    total = B * T

    w1f = w1.transpose(0, 2, 1).reshape(w1.shape[0], -1)
    w2f = w2.transpose(0, 2, 1).reshape(w2.shape[0], -1)
    w3f = jnp.pad(w3[:, :, 0], ((0, 5), (0, 0)))
    wpf = jnp.pad(wp[:, :, 0], ((0, 0), (0, 5)))
    b1c = b1[:, None]
    b2c = b2[:, None]
    b3c = jnp.pad(b3[:, None], ((0, 5), (0, 0)))
    bpc = bp[:, None]
    half = jnp.array([[3.5], [2.5], [2.0], [0.], [0.], [0.], [0.], [0.]],
                     jnp.float32)
    inv = jnp.array([[2.0 / 7.0], [2.0 / 5.0], [0.5], [0.], [0.], [0.],
                     [0.], [0.]], jnp.float32)

    body = functools.partial(_fsq_body, nsteps=nsteps, total=float(total))
    codes, out, probs32x8, perp11 = pl.pallas_call(
        body,
        grid=(nsteps,),
        in_specs=[
            pl.BlockSpec((NB, ci, T), lambda i: (i, 0, 0)),
            pl.BlockSpec(w1f.shape, lambda i: (0, 0)),
            pl.BlockSpec(b1c.shape, lambda i: (0, 0)),
            pl.BlockSpec(w2f.shape, lambda i: (0, 0)),
            pl.BlockSpec(b2c.shape, lambda i: (0, 0)),
            pl.BlockSpec(w3f.shape, lambda i: (0, 0)),
            pl.BlockSpec(b3c.shape, lambda i: (0, 0)),
            pl.BlockSpec(wpf.shape, lambda i: (0, 0)),
            pl.BlockSpec(bpc.shape, lambda i: (0, 0)),
            pl.BlockSpec((8, 1), lambda i: (0, 0)),
            pl.BlockSpec((8, 1), lambda i: (0, 0)),
        ],
        out_specs=[
            pl.BlockSpec((NB, T), lambda i: (i, 0)),
            pl.BlockSpec((NB, 64, T), lambda i: (i, 0, 0)),
            pl.BlockSpec((32, 8), lambda i: (0, 0)),
            pl.BlockSpec((1, 1), lambda i: (0, 0)),
        ],
        out_shape=[
            jax.ShapeDtypeStruct((B, T), jnp.int32),
            jax.ShapeDtypeStruct((B, 64, T), jnp.float32),
            jax.ShapeDtypeStruct((32, 8), jnp.float32),
            jax.ShapeDtypeStruct((1, 1), jnp.float32),
        ],
        scratch_shapes=[pltpu.VMEM((32, 8), jnp.float32)],
        compiler_params=pltpu.CompilerParams(
            dimension_semantics=("arbitrary",)),
    )(xr, w1f, b1c, w2f, b2c, w3f, b3c, wpf, bpc, half, inv)

    probs = probs32x8.reshape(-1)[:240]
    perplexity = perp11.reshape(())
    return codes, out, perplexity, probs


# speedup vs baseline: 1.7863x; 1.4434x over previous
import functools

import jax
import jax.numpy as jnp
from jax import lax
from jax.experimental import pallas as pl
from jax.experimental.pallas import tpu as pltpu

LEVELS = (8, 6, 5)
NB = 8


def _shift3(a, zcol):
    am = jnp.concatenate([zcol, a[:, :-1]], axis=1)
    ap = jnp.concatenate([a[:, 1:], zcol], axis=1)
    return jnp.concatenate([am, a, ap], axis=0)


def _fsq_body(x_ref, w1_ref, b1_ref, w2_ref, b2_ref, w3_ref, b3_ref,
              wp_ref, bp_ref, half_ref, inv_ref, codes_ref, out_ref,
              probs_ref, perp_ref, acc_ref, *, nsteps, total):
    step = pl.program_id(0)

    @pl.when(step == 0)
    def _init():
        acc_ref[:, :] = jnp.zeros((32, 8), jnp.float32)

    T = x_ref.shape[-1]
    ci = x_ref.shape[1] * x_ref.shape[2]
    half = half_ref[:, :]
    inv = inv_ref[:, :]

    wpf = wp_ref[:, :]
    acc = jnp.zeros((32, 8), jnp.float32)
    code_rows = []
    for b in range(NB):
        xb = x_ref[b].reshape(ci, T)
        x3 = _shift3(xb, jnp.zeros((ci, 1), jnp.float32))
        h1 = jnp.maximum(
            jnp.dot(w1_ref[:, :], x3, preferred_element_type=jnp.float32)
            + b1_ref[:, :], 0.0)
        h3 = _shift3(h1, jnp.zeros((h1.shape[0], 1), jnp.float32))
        h2 = jnp.maximum(
            jnp.dot(w2_ref[:, :], h3, preferred_element_type=jnp.float32)
            + b2_ref[:, :], 0.0)
        f = jnp.dot(w3_ref[:, :], h2,
                    preferred_element_type=jnp.float32) + b3_ref[:, :]
        tb = jnp.tanh(f)
        idxf = jnp.round((tb + 1.0) * half)
        q = idxf * inv - 1.0
        ob = bp_ref[:, :]
        for d in range(3):
            ob = ob + wpf[:, d:d + 1] * q[d:d + 1, :]
        out_ref[b] = ob
        idxi = idxf.astype(jnp.int32)
        i0 = idxi[0:1]
        code_rows.append(i0 + idxi[1:2] * 8 + idxi[2:3] * 48)
        j12 = idxi[1:2] + idxi[2:3] * 6
        oh8 = (lax.broadcasted_iota(jnp.int32, (8, T), 0) == i0
               ).astype(jnp.bfloat16)
        oh30 = (lax.broadcasted_iota(jnp.int32, (32, T), 0) == j12
                ).astype(jnp.bfloat16)
        acc = acc + lax.dot_general(
            oh30, oh8, (((1,), (1,)), ((), ())),
            preferred_element_type=jnp.float32)
    codes_ref[:, :] = jnp.concatenate(code_rows, axis=0)
    acc_ref[:, :] += acc

    @pl.when(step == nsteps - 1)
    def _fin():
        counts = acc_ref[:, :]
        probs = counts * (1.0 / total)
        pos = probs > 0
        ent = -jnp.sum(jnp.where(pos, probs * jnp.log(
            jnp.where(pos, probs, 1.0)), 0.0))
        probs_ref[:, :] = probs
        perp_ref[:, :] = jnp.maximum(jnp.exp(ent), 1.0)[None, None]


def kernel(x, w1, b1, w2, b2, w3, b3, wp, bp):
    B, C, S, T = x.shape
    ci = C * S
    nsteps = B // NB
    total = B * T

    w1f = w1.transpose(0, 2, 1).reshape(w1.shape[0], -1)
    w2f = w2.transpose(0, 2, 1).reshape(w2.shape[0], -1)
    w3f = jnp.pad(w3[:, :, 0], ((0, 5), (0, 0)))
    wpf = jnp.pad(wp[:, :, 0], ((0, 0), (0, 5)))
    b1c = b1[:, None]
    b2c = b2[:, None]
    b3c = jnp.pad(b3[:, None], ((0, 5), (0, 0)))
    bpc = bp[:, None]
    half = jnp.array([[3.5], [2.5], [2.0], [0.], [0.], [0.], [0.], [0.]],
                     jnp.float32)
    inv = jnp.array([[2.0 / 7.0], [2.0 / 5.0], [0.5], [0.], [0.], [0.],
                     [0.], [0.]], jnp.float32)

    body = functools.partial(_fsq_body, nsteps=nsteps, total=float(total))
    codes, out, probs32x8, perp11 = pl.pallas_call(
        body,
        grid=(nsteps,),
        in_specs=[
            pl.BlockSpec((NB, C, S, T), lambda i: (i, 0, 0, 0)),
            pl.BlockSpec(w1f.shape, lambda i: (0, 0)),
            pl.BlockSpec(b1c.shape, lambda i: (0, 0)),
            pl.BlockSpec(w2f.shape, lambda i: (0, 0)),
            pl.BlockSpec(b2c.shape, lambda i: (0, 0)),
            pl.BlockSpec(w3f.shape, lambda i: (0, 0)),
            pl.BlockSpec(b3c.shape, lambda i: (0, 0)),
            pl.BlockSpec(wpf.shape, lambda i: (0, 0)),
            pl.BlockSpec(bpc.shape, lambda i: (0, 0)),
            pl.BlockSpec((8, 1), lambda i: (0, 0)),
            pl.BlockSpec((8, 1), lambda i: (0, 0)),
        ],
        out_specs=[
            pl.BlockSpec((NB, T), lambda i: (i, 0)),
            pl.BlockSpec((NB, 64, T), lambda i: (i, 0, 0)),
            pl.BlockSpec((32, 8), lambda i: (0, 0)),
            pl.BlockSpec((1, 1), lambda i: (0, 0)),
        ],
        out_shape=[
            jax.ShapeDtypeStruct((B, T), jnp.int32),
            jax.ShapeDtypeStruct((B, 64, T), jnp.float32),
            jax.ShapeDtypeStruct((32, 8), jnp.float32),
            jax.ShapeDtypeStruct((1, 1), jnp.float32),
        ],
        scratch_shapes=[pltpu.VMEM((32, 8), jnp.float32)],
        compiler_params=pltpu.CompilerParams(
            dimension_semantics=("arbitrary",)),
    )(x, w1f, b1c, w2f, b2c, w3f, b3c, wpf, bpc, half, inv)

    probs = probs32x8.reshape(-1)[:240]
    perplexity = perp11.reshape(())
    return codes, out, perplexity, probs


# speedup vs baseline: 1.8259x; 1.0222x over previous
import functools

import jax
import jax.numpy as jnp
from jax import lax
from jax.experimental import pallas as pl
from jax.experimental.pallas import tpu as pltpu

LEVELS = (8, 6, 5)
NB = 16


def _shift3(a, zcol):
    am = jnp.concatenate([zcol, a[:, :-1]], axis=1)
    ap = jnp.concatenate([a[:, 1:], zcol], axis=1)
    return jnp.concatenate([am, a, ap], axis=0)


def _fsq_body(x_ref, w1_ref, b1_ref, w2_ref, b2_ref, w3_ref, b3_ref,
              wp_ref, bp_ref, half_ref, inv_ref, codes_ref, out_ref,
              probs_ref, perp_ref, acc_ref, *, nsteps, total):
    step = pl.program_id(0)

    @pl.when(step == 0)
    def _init():
        acc_ref[:, :] = jnp.zeros((32, 8), jnp.float32)

    T = x_ref.shape[-1]
    ci = x_ref.shape[1] * x_ref.shape[2]
    half = half_ref[:, :]
    inv = inv_ref[:, :]

    wpf = wp_ref[:, :]
    acc = jnp.zeros((32, 8), jnp.float32)
    code_rows = []
    for b in range(NB):
        xb = x_ref[b].reshape(ci, T)
        x3 = _shift3(xb, jnp.zeros((ci, 1), jnp.float32))
        h1 = jnp.maximum(
            jnp.dot(w1_ref[:, :], x3, preferred_element_type=jnp.float32)
            + b1_ref[:, :], 0.0)
        h3 = _shift3(h1, jnp.zeros((h1.shape[0], 1), jnp.float32))
        h2 = jnp.maximum(
            jnp.dot(w2_ref[:, :], h3, preferred_element_type=jnp.float32)
            + b2_ref[:, :], 0.0)
        f = jnp.dot(w3_ref[:, :], h2,
                    preferred_element_type=jnp.float32) + b3_ref[:, :]
        tb = jnp.tanh(f)
        idxf = jnp.round((tb + 1.0) * half)
        q = idxf * inv - 1.0
        ob = jnp.dot(wpf, q, preferred_element_type=jnp.float32) + bp_ref[:, :]
        out_ref[b] = ob
        idxi = idxf.astype(jnp.int32)
        i0 = idxi[0:1]
        code_rows.append(i0 + idxi[1:2] * 8 + idxi[2:3] * 48)
        j12 = idxi[1:2] + idxi[2:3] * 6
        oh8 = (lax.broadcasted_iota(jnp.int32, (8, T), 0) == i0
               ).astype(jnp.bfloat16)
        oh30 = (lax.broadcasted_iota(jnp.int32, (32, T), 0) == j12
                ).astype(jnp.bfloat16)
        acc = acc + lax.dot_general(
            oh30, oh8, (((1,), (1,)), ((), ())),
            preferred_element_type=jnp.float32)
    codes_ref[:, :] = jnp.concatenate(code_rows, axis=0)
    acc_ref[:, :] += acc

    @pl.when(step == nsteps - 1)
    def _fin():
        counts = acc_ref[:, :]
        probs = counts * (1.0 / total)
        pos = probs > 0
        ent = -jnp.sum(jnp.where(pos, probs * jnp.log(
            jnp.where(pos, probs, 1.0)), 0.0))
        probs_ref[:, :] = probs
        perp_ref[:, :] = jnp.maximum(jnp.exp(ent), 1.0)[None, None]


def kernel(x, w1, b1, w2, b2, w3, b3, wp, bp):
    B, C, S, T = x.shape
    ci = C * S
    nsteps = B // NB
    total = B * T

    w1f = w1.transpose(0, 2, 1).reshape(w1.shape[0], -1)
    w2f = w2.transpose(0, 2, 1).reshape(w2.shape[0], -1)
    w3f = jnp.pad(w3[:, :, 0], ((0, 5), (0, 0)))
    wpf = jnp.pad(wp[:, :, 0], ((0, 0), (0, 5)))
    b1c = b1[:, None]
    b2c = b2[:, None]
    b3c = jnp.pad(b3[:, None], ((0, 5), (0, 0)))
    bpc = bp[:, None]
    half = jnp.array([[3.5], [2.5], [2.0], [0.], [0.], [0.], [0.], [0.]],
                     jnp.float32)
    inv = jnp.array([[2.0 / 7.0], [2.0 / 5.0], [0.5], [0.], [0.], [0.],
                     [0.], [0.]], jnp.float32)

    body = functools.partial(_fsq_body, nsteps=nsteps, total=float(total))
    codes, out, probs32x8, perp11 = pl.pallas_call(
        body,
        grid=(nsteps,),
        in_specs=[
            pl.BlockSpec((NB, C, S, T), lambda i: (i, 0, 0, 0)),
            pl.BlockSpec(w1f.shape, lambda i: (0, 0)),
            pl.BlockSpec(b1c.shape, lambda i: (0, 0)),
            pl.BlockSpec(w2f.shape, lambda i: (0, 0)),
            pl.BlockSpec(b2c.shape, lambda i: (0, 0)),
            pl.BlockSpec(w3f.shape, lambda i: (0, 0)),
            pl.BlockSpec(b3c.shape, lambda i: (0, 0)),
            pl.BlockSpec(wpf.shape, lambda i: (0, 0)),
            pl.BlockSpec(bpc.shape, lambda i: (0, 0)),
            pl.BlockSpec((8, 1), lambda i: (0, 0)),
            pl.BlockSpec((8, 1), lambda i: (0, 0)),
        ],
        out_specs=[
            pl.BlockSpec((NB, T), lambda i: (i, 0)),
            pl.BlockSpec((NB, 64, T), lambda i: (i, 0, 0)),
            pl.BlockSpec((32, 8), lambda i: (0, 0)),
            pl.BlockSpec((1, 1), lambda i: (0, 0)),
        ],
        out_shape=[
            jax.ShapeDtypeStruct((B, T), jnp.int32),
            jax.ShapeDtypeStruct((B, 64, T), jnp.float32),
            jax.ShapeDtypeStruct((32, 8), jnp.float32),
            jax.ShapeDtypeStruct((1, 1), jnp.float32),
        ],
        scratch_shapes=[pltpu.VMEM((32, 8), jnp.float32)],
        compiler_params=pltpu.CompilerParams(
            dimension_semantics=("arbitrary",)),
    )(x, w1f, b1c, w2f, b2c, w3f, b3c, wpf, bpc, half, inv)

    probs = probs32x8.reshape(-1)[:240]
    perplexity = perp11.reshape(())
    return codes, out, perplexity, probs
